# Initial kernel scaffold; baseline (speedup 1.0000x reference)
import jax
import jax.numpy as jnp
from jax.experimental import pallas as pl


def kernel(concept_table, relation_table, W_ih_f, W_hh_f, b_ih_f, b_hh_f, W_ih_r, W_hh_r, b_ih_r, b_hh_r, W_hts, b_hts, W_fc, b_fc, cpt_paths, rel_paths, ht_ids, path_len):
    raise NotImplementedError("write your pallas kernel here")


# speedup vs baseline: 6.0204x; 6.0204x over previous
import functools

import jax
import jax.numpy as jnp
from jax import lax
from jax.experimental import pallas as pl
from jax.experimental.pallas import tpu as pltpu
from jax.experimental.pallas import tpu_sc as plsc

_NC = 2
_NS = 16
_NW = _NC * _NS
_CHUNK = 128


def _sc_gather_body(n_chunks, idx2d_hbm, table_hbm, out_hbm, idx_v, rows_v, sem):
    wid = lax.axis_index("s") * _NC + lax.axis_index("c")
    row0 = wid * n_chunks
    pltpu.sync_copy(idx2d_hbm.at[pl.ds(row0, n_chunks)], idx_v)

    def body(j, carry):
        pltpu.async_copy(table_hbm.at[idx_v.at[j]], rows_v, sem).wait()
        pltpu.sync_copy(rows_v, out_hbm.at[pl.ds((row0 + j) * _CHUNK, _CHUNK)])
        return carry

    lax.fori_loop(0, n_chunks, body, 0)


def _sc_gather(idx_flat, table):
    B = idx_flat.shape[0]
    D = table.shape[1]
    assert B % (_NW * _CHUNK) == 0
    n_chunks = B // (_NW * _CHUNK)
    idx2d = idx_flat.reshape(-1, _CHUNK)
    mesh = plsc.VectorSubcoreMesh(core_axis_name="c", subcore_axis_name="s")
    k = functools.partial(
        pl.kernel,
        mesh=mesh,
        out_type=jax.ShapeDtypeStruct((B, D), table.dtype),
        scratch_types=[
            pltpu.VMEM((n_chunks, _CHUNK), jnp.int32),
            pltpu.VMEM((_CHUNK, D), table.dtype),
            pltpu.SemaphoreType.DMA,
        ],
    )(functools.partial(_sc_gather_body, n_chunks))
    return k(idx2d, table)


def _enc_body(T, cpt_ref, meta_ref, ht_ref, rel32_ref, wc_ref, wr_ref,
              whf_ref, whr_ref, bf_ref, br_ref, whts_ref, bhts_ref,
              wfh_ref, wff_ref, wfr_ref, bfc_ref, out_ref):
    f32 = jnp.float32
    Bp = out_ref.shape[0]
    H = whf_ref.shape[0]

    rel_proj = jnp.dot(rel32_ref[...], wr_ref[...], preferred_element_type=f32)

    gx = []
    for t in range(T):
        x = cpt_ref[t]
        r = meta_ref[t]
        r = jnp.where(r >= 17, r - 17, r)
        oh = (r[:, None] == lax.broadcasted_iota(jnp.int32, (Bp, 32), 1)
              ).astype(f32)
        g = (jnp.dot(x, wc_ref[...], preferred_element_type=f32)
             + jnp.dot(oh, rel_proj, preferred_element_type=f32))
        gx.append(g)

    last = (jnp.clip(meta_ref[T], 1, T) - 1)[:, None]

    def lstm(wh_ref, b_ref, col0, order):
        h = jnp.zeros((Bp, H), f32)
        c = jnp.zeros((Bp, H), f32)
        sel = jnp.zeros((Bp, H), f32)
        for t in order:
            g = (gx[t][:, col0:col0 + 4 * H]
                 + jnp.dot(h, wh_ref[...], preferred_element_type=f32)
                 + b_ref[...])
            i_g = jax.nn.sigmoid(g[:, 0:H])
            f_g = jax.nn.sigmoid(g[:, H:2 * H])
            g_g = jnp.tanh(g[:, 2 * H:3 * H])
            o_g = jax.nn.sigmoid(g[:, 3 * H:4 * H])
            c = f_g * c + i_g * g_g
            h = o_g * jnp.tanh(c)
            sel = sel + (last == t).astype(f32) * h
        return sel

    sel_f = lstm(whf_ref, bf_ref, 0, range(T))
    sel_r = lstm(whr_ref, br_ref, 4 * H, range(T - 1, -1, -1))

    def leaky(x):
        return jnp.where(x >= 0, x, 0.01 * x)

    hts = leaky(jnp.dot(ht_ref[...], whts_ref[...], preferred_element_type=f32)
                + bhts_ref[...])
    fin = (jnp.dot(hts, wfh_ref[...], preferred_element_type=f32)
           + jnp.dot(sel_f, wff_ref[...], preferred_element_type=f32)
           + jnp.dot(sel_r, wfr_ref[...], preferred_element_type=f32)
           + bfc_ref[...])
    out_ref[...] = leaky(fin)


def _encode(cpt_emb, meta, ht_vecs, rel32, wc, wr, whf, whr, bf, br,
            whts, bhts, wfh, wff, wfr, bfc, block_p=512):
    T, P, D = cpt_emb.shape
    O = wfh.shape[1]
    grid = (P // block_p,)
    full = lambda shape: pl.BlockSpec(shape, lambda i: (0,) * len(shape))
    return pl.pallas_call(
        functools.partial(_enc_body, T),
        grid=grid,
        in_specs=[
            pl.BlockSpec((T, block_p, D), lambda i: (0, i, 0)),
            pl.BlockSpec((8, block_p), lambda i: (0, i)),
            pl.BlockSpec((block_p, 2 * D), lambda i: (i, 0)),
            full(rel32.shape), full(wc.shape), full(wr.shape),
            full(whf.shape), full(whr.shape), full(bf.shape), full(br.shape),
            full(whts.shape), full(bhts.shape),
            full(wfh.shape), full(wff.shape), full(wfr.shape), full(bfc.shape),
        ],
        out_specs=pl.BlockSpec((block_p, O), lambda i: (i, 0)),
        out_shape=jax.ShapeDtypeStruct((P, O), jnp.float32),
        compiler_params=pltpu.CompilerParams(
            dimension_semantics=("arbitrary",)),
    )(cpt_emb, meta, ht_vecs, rel32, wc, wr, whf, whr, bf, br,
      whts, bhts, wfh, wff, wfr, bfc)


def kernel(concept_table, relation_table, W_ih_f, W_hh_f, b_ih_f, b_hh_f,
           W_ih_r, W_hh_r, b_ih_r, b_hh_r, W_hts, b_hts, W_fc, b_fc,
           cpt_paths, rel_paths, ht_ids, path_len):
    P, T = cpt_paths.shape
    V, D = concept_table.shape
    H = W_hh_f.shape[1]

    cpt_idx = cpt_paths.astype(jnp.int32).T.reshape(-1)
    ht_idx = ht_ids.astype(jnp.int32).reshape(-1)
    idx_all = jnp.concatenate([cpt_idx, ht_idx])
    gathered = _sc_gather(idx_all, concept_table)
    cpt_emb = gathered[:T * P].reshape(T, P, D)
    ht_vecs = gathered[T * P:].reshape(P, 2 * D)

    meta = jnp.concatenate([
        rel_paths.astype(jnp.int32).T,
        path_len.astype(jnp.int32)[None],
        jnp.zeros((8 - T - 1, P), jnp.int32),
    ], axis=0)
    rel32 = jnp.zeros((32, D), jnp.float32).at[:relation_table.shape[0]].set(
        relation_table)
    wc = jnp.concatenate([W_ih_f[:, :D].T, W_ih_r[:, :D].T], axis=1)
    wr = jnp.concatenate([W_ih_f[:, D:].T, W_ih_r[:, D:].T], axis=1)
    whf = W_hh_f.T
    whr = W_hh_r.T
    bf = (b_ih_f + b_hh_f)[None]
    br = (b_ih_r + b_hh_r)[None]
    wfh = W_fc[:2 * H]
    wff = W_fc[2 * H:3 * H]
    wfr = W_fc[3 * H:]

    return _encode(cpt_emb, meta, ht_vecs, rel32, wc, wr, whf, whr, bf, br,
                   W_hts, b_hts[None], wfh, wff, wfr, b_fc[None])


# speedup vs baseline: 6.0432x; 1.0038x over previous
import functools

import jax
import jax.numpy as jnp
from jax import lax
from jax.experimental import pallas as pl
from jax.experimental.pallas import tpu as pltpu
from jax.experimental.pallas import tpu_sc as plsc

_NC = 2
_NS = 16
_NW = _NC * _NS
_CHUNK = 128


def _sc_gather_body(n_chunks, idx2d_hbm, table_hbm, out_hbm, idx_v, rows_v, sem):
    wid = lax.axis_index("s") * _NC + lax.axis_index("c")
    row0 = wid * n_chunks
    pltpu.sync_copy(idx2d_hbm.at[pl.ds(row0, n_chunks)], idx_v)

    def body(j, carry):
        pltpu.async_copy(table_hbm.at[idx_v.at[j]], rows_v, sem).wait()
        pltpu.sync_copy(rows_v, out_hbm.at[pl.ds((row0 + j) * _CHUNK, _CHUNK)])
        return carry

    lax.fori_loop(0, n_chunks, body, 0)


def _sc_gather(idx_flat, table):
    B = idx_flat.shape[0]
    D = table.shape[1]
    assert B % (_NW * _CHUNK) == 0
    n_chunks = B // (_NW * _CHUNK)
    idx2d = idx_flat.reshape(-1, _CHUNK)
    mesh = plsc.VectorSubcoreMesh(core_axis_name="c", subcore_axis_name="s")
    k = functools.partial(
        pl.kernel,
        mesh=mesh,
        out_type=jax.ShapeDtypeStruct((B, D), table.dtype),
        scratch_types=[
            pltpu.VMEM((n_chunks, _CHUNK), jnp.int32),
            pltpu.VMEM((_CHUNK, D), table.dtype),
            pltpu.SemaphoreType.DMA,
        ],
    )(functools.partial(_sc_gather_body, n_chunks))
    return k(idx2d, table)


def _enc_body(T, cpt_ref, meta_ref, ht_ref, rel32_ref, wc_ref, wr_ref,
              whf_ref, whr_ref, bf_ref, br_ref, whts_ref, bhts_ref,
              wfh_ref, wff_ref, wfr_ref, bfc_ref, out_ref):
    f32 = jnp.float32
    bf16 = jnp.bfloat16
    Bp = out_ref.shape[0]
    H = whf_ref.shape[0]

    rel_proj = jnp.dot(rel32_ref[...], wr_ref[...],
                       preferred_element_type=f32).astype(bf16)

    gx = []
    for t in range(T):
        x = cpt_ref[t].astype(bf16)
        r = meta_ref[t]
        r = jnp.where(r >= 17, r - 17, r)
        oh = (r[:, None] == lax.broadcasted_iota(jnp.int32, (Bp, 32), 1)
              ).astype(bf16)
        g = (jnp.dot(x, wc_ref[...], preferred_element_type=f32)
             + jnp.dot(oh, rel_proj, preferred_element_type=f32))
        gx.append(g)

    last = (jnp.clip(meta_ref[T], 1, T) - 1)[:, None]

    def lstm(wh_ref, b_ref, col0, order):
        h = jnp.zeros((Bp, H), f32)
        c = jnp.zeros((Bp, H), f32)
        sel = jnp.zeros((Bp, H), f32)
        for t in order:
            g = (gx[t][:, col0:col0 + 4 * H]
                 + jnp.dot(h.astype(bf16), wh_ref[...],
                           preferred_element_type=f32)
                 + b_ref[...])
            i_g = jax.nn.sigmoid(g[:, 0:H])
            f_g = jax.nn.sigmoid(g[:, H:2 * H])
            g_g = jnp.tanh(g[:, 2 * H:3 * H])
            o_g = jax.nn.sigmoid(g[:, 3 * H:4 * H])
            c = f_g * c + i_g * g_g
            h = o_g * jnp.tanh(c)
            sel = sel + (last == t).astype(f32) * h
        return sel

    sel_f = lstm(whf_ref, bf_ref, 0, range(T))
    sel_r = lstm(whr_ref, br_ref, 4 * H, range(T - 1, -1, -1))

    def leaky(x):
        return jnp.where(x >= 0, x, 0.01 * x)

    hts = leaky(jnp.dot(ht_ref[...].astype(bf16), whts_ref[...],
                        preferred_element_type=f32) + bhts_ref[...])
    fin = (jnp.dot(hts.astype(bf16), wfh_ref[...], preferred_element_type=f32)
           + jnp.dot(sel_f.astype(bf16), wff_ref[...],
                     preferred_element_type=f32)
           + jnp.dot(sel_r.astype(bf16), wfr_ref[...],
                     preferred_element_type=f32)
           + bfc_ref[...])
    out_ref[...] = leaky(fin)


def _encode(cpt_emb, meta, ht_vecs, rel32, wc, wr, whf, whr, bf, br,
            whts, bhts, wfh, wff, wfr, bfc, block_p=512):
    T, P, D = cpt_emb.shape
    O = wfh.shape[1]
    grid = (P // block_p,)
    full = lambda shape: pl.BlockSpec(shape, lambda i: (0,) * len(shape))
    return pl.pallas_call(
        functools.partial(_enc_body, T),
        grid=grid,
        in_specs=[
            pl.BlockSpec((T, block_p, D), lambda i: (0, i, 0)),
            pl.BlockSpec((8, block_p), lambda i: (0, i)),
            pl.BlockSpec((block_p, 2 * D), lambda i: (i, 0)),
            full(rel32.shape), full(wc.shape), full(wr.shape),
            full(whf.shape), full(whr.shape), full(bf.shape), full(br.shape),
            full(whts.shape), full(bhts.shape),
            full(wfh.shape), full(wff.shape), full(wfr.shape), full(bfc.shape),
        ],
        out_specs=pl.BlockSpec((block_p, O), lambda i: (i, 0)),
        out_shape=jax.ShapeDtypeStruct((P, O), jnp.float32),
        compiler_params=pltpu.CompilerParams(
            dimension_semantics=("arbitrary",)),
    )(cpt_emb, meta, ht_vecs, rel32, wc, wr, whf, whr, bf, br,
      whts, bhts, wfh, wff, wfr, bfc)


def kernel(concept_table, relation_table, W_ih_f, W_hh_f, b_ih_f, b_hh_f,
           W_ih_r, W_hh_r, b_ih_r, b_hh_r, W_hts, b_hts, W_fc, b_fc,
           cpt_paths, rel_paths, ht_ids, path_len):
    P, T = cpt_paths.shape
    V, D = concept_table.shape
    H = W_hh_f.shape[1]

    cpt_idx = cpt_paths.astype(jnp.int32).T.reshape(-1)
    ht_idx = ht_ids.astype(jnp.int32).reshape(-1)
    idx_all = jnp.concatenate([cpt_idx, ht_idx])
    gathered = _sc_gather(idx_all, concept_table)
    cpt_emb = gathered[:T * P].reshape(T, P, D)
    ht_vecs = gathered[T * P:].reshape(P, 2 * D)

    meta = jnp.concatenate([
        rel_paths.astype(jnp.int32).T,
        path_len.astype(jnp.int32)[None],
        jnp.zeros((8 - T - 1, P), jnp.int32),
    ], axis=0)
    bf16 = jnp.bfloat16
    rel32 = jnp.zeros((32, D), jnp.float32).at[:relation_table.shape[0]].set(
        relation_table).astype(bf16)
    wc = jnp.concatenate([W_ih_f[:, :D].T, W_ih_r[:, :D].T], axis=1).astype(bf16)
    wr = jnp.concatenate([W_ih_f[:, D:].T, W_ih_r[:, D:].T], axis=1).astype(bf16)
    whf = W_hh_f.T.astype(bf16)
    whr = W_hh_r.T.astype(bf16)
    bf = (b_ih_f + b_hh_f)[None]
    br = (b_ih_r + b_hh_r)[None]
    wfh = W_fc[:2 * H].astype(bf16)
    wff = W_fc[2 * H:3 * H].astype(bf16)
    wfr = W_fc[3 * H:].astype(bf16)

    return _encode(cpt_emb, meta, ht_vecs, rel32, wc, wr, whf, whr, bf, br,
                   W_hts.astype(bf16), b_hts[None], wfh, wff, wfr, b_fc[None])


# speedup vs baseline: 7.2838x; 1.2053x over previous
import functools

import jax
import jax.numpy as jnp
from jax import lax
from jax.experimental import pallas as pl
from jax.experimental.pallas import tpu as pltpu
from jax.experimental.pallas import tpu_sc as plsc

_NC = 2
_NS = 16
_NW = _NC * _NS
_CHUNK = 128


def _sc_gather_body(n_chunks, c1, idx2d_hbm, table_hbm, out1_hbm, out2_hbm,
                    idx_v, rows_v, sem):
    wid = lax.axis_index("s") * _NC + lax.axis_index("c")
    row0 = wid * n_chunks
    pltpu.sync_copy(idx2d_hbm.at[pl.ds(row0, n_chunks)], idx_v)

    def gather(j):
        b = lax.rem(j, 2)
        pltpu.async_copy(table_hbm.at[idx_v.at[j]], rows_v.at[b], sem)

    gather(0)

    def body(j, carry):
        @pl.when(j + 1 < n_chunks)
        def _():
            gather(j + 1)

        b = lax.rem(j, 2)
        pltpu.make_async_copy(table_hbm.at[idx_v.at[j]], rows_v.at[b],
                              sem).wait()
        c = row0 + j

        @pl.when(c < c1)
        def _():
            pltpu.sync_copy(rows_v.at[b],
                            out1_hbm.at[pl.ds(c * _CHUNK, _CHUNK)])

        @pl.when(c >= c1)
        def _():
            pltpu.sync_copy(rows_v.at[b],
                            out2_hbm.at[pl.ds((c - c1) * _CHUNK, _CHUNK)])

        return carry

    lax.fori_loop(0, n_chunks, body, 0)


def _sc_gather(idx_flat, table, n1):
    B = idx_flat.shape[0]
    D = table.shape[1]
    assert B % (_NW * _CHUNK) == 0 and n1 % _CHUNK == 0
    n_chunks = B // (_NW * _CHUNK)
    idx2d = idx_flat.reshape(-1, _CHUNK)
    mesh = plsc.VectorSubcoreMesh(core_axis_name="c", subcore_axis_name="s")
    k = functools.partial(
        pl.kernel,
        mesh=mesh,
        out_type=(
            jax.ShapeDtypeStruct((n1, D), table.dtype),
            jax.ShapeDtypeStruct((B - n1, D), table.dtype),
        ),
        scratch_types=[
            pltpu.VMEM((n_chunks, _CHUNK), jnp.int32),
            pltpu.VMEM((2, _CHUNK, D), table.dtype),
            pltpu.SemaphoreType.DMA,
        ],
    )(functools.partial(_sc_gather_body, n_chunks, n1 // _CHUNK))
    return k(idx2d, table)


def _enc_body(T, cpt_ref, meta_ref, ht_ref, rel32_ref, wc_ref, wr_ref,
              whf_ref, whr_ref, bf_ref, br_ref, whts_ref, bhts_ref,
              wfh_ref, wff_ref, wfr_ref, bfc_ref, out_ref):
    f32 = jnp.float32
    bf16 = jnp.bfloat16
    Bp = out_ref.shape[0]
    H = whf_ref.shape[0]

    rel_proj = jnp.dot(rel32_ref[...], wr_ref[...],
                       preferred_element_type=f32).astype(bf16)

    gx = []
    for t in range(T):
        x = cpt_ref[t].astype(bf16)
        r = meta_ref[t]
        r = jnp.where(r >= 17, r - 17, r)
        oh = (r[:, None] == lax.broadcasted_iota(jnp.int32, (Bp, 32), 1)
              ).astype(bf16)
        g = (jnp.dot(x, wc_ref[...], preferred_element_type=f32)
             + jnp.dot(oh, rel_proj, preferred_element_type=f32))
        gx.append(g)

    last = (jnp.clip(meta_ref[T], 1, T) - 1)[:, None]

    def lstm(wh_ref, b_ref, col0, order):
        h = jnp.zeros((Bp, H), f32)
        c = jnp.zeros((Bp, H), f32)
        sel = jnp.zeros((Bp, H), f32)
        for t in order:
            g = (gx[t][:, col0:col0 + 4 * H]
                 + jnp.dot(h.astype(bf16), wh_ref[...],
                           preferred_element_type=f32)
                 + b_ref[...])
            i_g = jax.nn.sigmoid(g[:, 0:H])
            f_g = jax.nn.sigmoid(g[:, H:2 * H])
            g_g = jnp.tanh(g[:, 2 * H:3 * H])
            o_g = jax.nn.sigmoid(g[:, 3 * H:4 * H])
            c = f_g * c + i_g * g_g
            h = o_g * jnp.tanh(c)
            sel = sel + (last == t).astype(f32) * h
        return sel

    sel_f = lstm(whf_ref, bf_ref, 0, range(T))
    sel_r = lstm(whr_ref, br_ref, 4 * H, range(T - 1, -1, -1))

    def leaky(x):
        return jnp.where(x >= 0, x, 0.01 * x)

    hts = leaky(jnp.dot(ht_ref[...].astype(bf16), whts_ref[...],
                        preferred_element_type=f32) + bhts_ref[...])
    fin = (jnp.dot(hts.astype(bf16), wfh_ref[...], preferred_element_type=f32)
           + jnp.dot(sel_f.astype(bf16), wff_ref[...],
                     preferred_element_type=f32)
           + jnp.dot(sel_r.astype(bf16), wfr_ref[...],
                     preferred_element_type=f32)
           + bfc_ref[...])
    out_ref[...] = leaky(fin)


def _encode(cpt_emb, meta, ht_vecs, rel32, wc, wr, whf, whr, bf, br,
            whts, bhts, wfh, wff, wfr, bfc, block_p=512):
    T, P, D = cpt_emb.shape
    O = wfh.shape[1]
    grid = (P // block_p,)
    full = lambda shape: pl.BlockSpec(shape, lambda i: (0,) * len(shape))
    return pl.pallas_call(
        functools.partial(_enc_body, T),
        grid=grid,
        in_specs=[
            pl.BlockSpec((T, block_p, D), lambda i: (0, i, 0)),
            pl.BlockSpec((8, block_p), lambda i: (0, i)),
            pl.BlockSpec((block_p, 2 * D), lambda i: (i, 0)),
            full(rel32.shape), full(wc.shape), full(wr.shape),
            full(whf.shape), full(whr.shape), full(bf.shape), full(br.shape),
            full(whts.shape), full(bhts.shape),
            full(wfh.shape), full(wff.shape), full(wfr.shape), full(bfc.shape),
        ],
        out_specs=pl.BlockSpec((block_p, O), lambda i: (i, 0)),
        out_shape=jax.ShapeDtypeStruct((P, O), jnp.float32),
        compiler_params=pltpu.CompilerParams(
            dimension_semantics=("arbitrary",)),
    )(cpt_emb, meta, ht_vecs, rel32, wc, wr, whf, whr, bf, br,
      whts, bhts, wfh, wff, wfr, bfc)


def kernel(concept_table, relation_table, W_ih_f, W_hh_f, b_ih_f, b_hh_f,
           W_ih_r, W_hh_r, b_ih_r, b_hh_r, W_hts, b_hts, W_fc, b_fc,
           cpt_paths, rel_paths, ht_ids, path_len):
    P, T = cpt_paths.shape
    V, D = concept_table.shape
    H = W_hh_f.shape[1]

    cpt_idx = cpt_paths.astype(jnp.int32).T.reshape(-1)
    ht_idx = ht_ids.astype(jnp.int32).reshape(-1)
    idx_all = jnp.concatenate([cpt_idx, ht_idx])
    cpt_rows, ht_rows = _sc_gather(idx_all, concept_table, T * P)
    cpt_emb = cpt_rows.reshape(T, P, D)
    ht_vecs = ht_rows.reshape(P, 2 * D)

    meta = jnp.concatenate([
        rel_paths.astype(jnp.int32).T,
        path_len.astype(jnp.int32)[None],
        jnp.zeros((8 - T - 1, P), jnp.int32),
    ], axis=0)
    bf16 = jnp.bfloat16
    rel32 = jnp.zeros((32, D), jnp.float32).at[:relation_table.shape[0]].set(
        relation_table).astype(bf16)
    wc = jnp.concatenate([W_ih_f[:, :D].T, W_ih_r[:, :D].T], axis=1).astype(bf16)
    wr = jnp.concatenate([W_ih_f[:, D:].T, W_ih_r[:, D:].T], axis=1).astype(bf16)
    whf = W_hh_f.T.astype(bf16)
    whr = W_hh_r.T.astype(bf16)
    bf = (b_ih_f + b_hh_f)[None]
    br = (b_ih_r + b_hh_r)[None]
    wfh = W_fc[:2 * H].astype(bf16)
    wff = W_fc[2 * H:3 * H].astype(bf16)
    wfr = W_fc[3 * H:].astype(bf16)

    return _encode(cpt_emb, meta, ht_vecs, rel32, wc, wr, whf, whr, bf, br,
                   W_hts.astype(bf16), b_hts[None], wfh, wff, wfr, b_fc[None])


# speedup vs baseline: 9.3333x; 1.2814x over previous
import functools

import jax
import jax.numpy as jnp
from jax import lax
from jax.experimental import pallas as pl
from jax.experimental.pallas import tpu as pltpu
from jax.experimental.pallas import tpu_sc as plsc

_NC = 2
_NS = 16
_NW = _NC * _NS
_CHUNK = 128


def _sc_gather_body(n_chunks, c1, idx2d_hbm, table_hbm, out1_hbm, out2_hbm,
                    idx_v, rows_v, sem):
    wid = lax.axis_index("s") * _NC + lax.axis_index("c")
    row0 = wid * n_chunks
    pltpu.sync_copy(idx2d_hbm.at[pl.ds(row0, n_chunks)], idx_v)

    def gather(j):
        b = lax.rem(j, 2)
        pltpu.async_copy(table_hbm.at[idx_v.at[j]], rows_v.at[b], sem)

    gather(0)

    def body(j, carry):
        @pl.when(j + 1 < n_chunks)
        def _():
            gather(j + 1)

        b = lax.rem(j, 2)
        pltpu.make_async_copy(table_hbm.at[idx_v.at[j]], rows_v.at[b],
                              sem).wait()
        c = row0 + j

        @pl.when(c < c1)
        def _():
            pltpu.sync_copy(rows_v.at[b],
                            out1_hbm.at[pl.ds(c * _CHUNK, _CHUNK)])

        @pl.when(c >= c1)
        def _():
            pltpu.sync_copy(rows_v.at[b],
                            out2_hbm.at[pl.ds((c - c1) * _CHUNK, _CHUNK)])

        return carry

    lax.fori_loop(0, n_chunks, body, 0)


def _sc_gather(idx_flat, table, n1):
    B = idx_flat.shape[0]
    D = table.shape[1]
    assert B % (_NW * _CHUNK) == 0 and n1 % _CHUNK == 0
    n_chunks = B // (_NW * _CHUNK)
    idx2d = idx_flat.reshape(-1, _CHUNK)
    mesh = plsc.VectorSubcoreMesh(core_axis_name="c", subcore_axis_name="s")
    k = functools.partial(
        pl.kernel,
        mesh=mesh,
        out_type=(
            jax.ShapeDtypeStruct((n1, D), table.dtype),
            jax.ShapeDtypeStruct((B - n1, D), table.dtype),
        ),
        scratch_types=[
            pltpu.VMEM((n_chunks, _CHUNK), jnp.int32),
            pltpu.VMEM((2, _CHUNK, D), table.dtype),
            pltpu.SemaphoreType.DMA,
        ],
    )(functools.partial(_sc_gather_body, n_chunks, n1 // _CHUNK))
    return k(idx2d, table)


def _enc_body(T, cpt_ref, meta_ref, ht_ref, rel128_ref, wc_ref, wr_ref,
              whf_ref, whr_ref, bf_ref, br_ref, whts_ref, bhts_ref,
              wfc_ref, bfc_ref, out_ref):
    f32 = jnp.float32
    bf16 = jnp.bfloat16
    Bp = out_ref.shape[0]
    H = whf_ref.shape[0]

    rel_proj = jnp.dot(rel128_ref[...], wr_ref[...],
                       preferred_element_type=f32).astype(bf16)
    wfull = jnp.concatenate([wc_ref[...], rel_proj], axis=0)

    gx = []
    for t in range(T):
        x = cpt_ref[t].astype(bf16)
        r = meta_ref[t]
        r = jnp.where(r >= 17, r - 17, r)
        oh = (r[:, None] == lax.broadcasted_iota(jnp.int32, (Bp, H), 1)
              ).astype(bf16)
        xcat = jnp.concatenate([x, oh], axis=1)
        gx.append(jnp.dot(xcat, wfull, preferred_element_type=f32))

    last = (jnp.clip(meta_ref[T], 1, T) - 1)[:, None]

    def sig2(x):
        return 0.5 * jnp.tanh(x) + 0.5

    def lstm(wh_ref, b_ref, col0, order):
        h = jnp.zeros((Bp, H), f32)
        c = jnp.zeros((Bp, H), f32)
        sel = jnp.zeros((Bp, H), f32)
        for t in order:
            g = (gx[t][:, col0:col0 + 4 * H]
                 + jnp.dot(h.astype(bf16), wh_ref[...],
                           preferred_element_type=f32)
                 + b_ref[...])
            i_g = sig2(g[:, 0:H])
            f_g = sig2(g[:, H:2 * H])
            g_g = jnp.tanh(g[:, 2 * H:3 * H])
            o_g = sig2(g[:, 3 * H:4 * H])
            c = f_g * c + i_g * g_g
            h = o_g * jnp.tanh(c)
            sel = sel + (last == t).astype(f32) * h
        return sel

    sel_f = lstm(whf_ref, bf_ref, 0, range(T))
    sel_r = lstm(whr_ref, br_ref, 4 * H, range(T - 1, -1, -1))

    def leaky(x):
        return jnp.where(x >= 0, x, 0.01 * x)

    hts = leaky(jnp.dot(ht_ref[...].astype(bf16), whts_ref[...],
                        preferred_element_type=f32) + bhts_ref[...])
    cat = jnp.concatenate([hts.astype(bf16), sel_f.astype(bf16),
                           sel_r.astype(bf16)], axis=1)
    fin = jnp.dot(cat, wfc_ref[...], preferred_element_type=f32) + bfc_ref[...]
    out_ref[...] = leaky(fin)


def _encode(cpt_emb, meta, ht_vecs, rel128, wc, wr, whf, whr, bf, br,
            whts, bhts, wfc, bfc, block_p=512):
    T, P, D = cpt_emb.shape
    O = wfc.shape[1]
    grid = (P // block_p,)
    full = lambda shape: pl.BlockSpec(shape, lambda i: (0,) * len(shape))
    return pl.pallas_call(
        functools.partial(_enc_body, T),
        grid=grid,
        in_specs=[
            pl.BlockSpec((T, block_p, D), lambda i: (0, i, 0)),
            pl.BlockSpec((8, block_p), lambda i: (0, i)),
            pl.BlockSpec((block_p, 2 * D), lambda i: (i, 0)),
            full(rel128.shape), full(wc.shape), full(wr.shape),
            full(whf.shape), full(whr.shape), full(bf.shape), full(br.shape),
            full(whts.shape), full(bhts.shape),
            full(wfc.shape), full(bfc.shape),
        ],
        out_specs=pl.BlockSpec((block_p, O), lambda i: (i, 0)),
        out_shape=jax.ShapeDtypeStruct((P, O), jnp.float32),
        compiler_params=pltpu.CompilerParams(
            dimension_semantics=("arbitrary",)),
    )(cpt_emb, meta, ht_vecs, rel128, wc, wr, whf, whr, bf, br,
      whts, bhts, wfc, bfc)


def kernel(concept_table, relation_table, W_ih_f, W_hh_f, b_ih_f, b_hh_f,
           W_ih_r, W_hh_r, b_ih_r, b_hh_r, W_hts, b_hts, W_fc, b_fc,
           cpt_paths, rel_paths, ht_ids, path_len):
    P, T = cpt_paths.shape
    V, D = concept_table.shape
    H = W_hh_f.shape[1]

    cpt_idx = cpt_paths.astype(jnp.int32).T.reshape(-1)
    ht_idx = ht_ids.astype(jnp.int32).reshape(-1)
    idx_all = jnp.concatenate([cpt_idx, ht_idx])
    cpt_rows, ht_rows = _sc_gather(idx_all, concept_table, T * P)
    cpt_emb = cpt_rows.reshape(T, P, D)
    ht_vecs = ht_rows.reshape(P, 2 * D)

    meta = jnp.concatenate([
        rel_paths.astype(jnp.int32).T,
        path_len.astype(jnp.int32)[None],
        jnp.zeros((8 - T - 1, P), jnp.int32),
    ], axis=0)
    bf16 = jnp.bfloat16
    s4 = jnp.concatenate([jnp.full((H,), 0.5), jnp.full((H,), 0.5),
                          jnp.ones((H,)), jnp.full((H,), 0.5)])[None]
    s8 = jnp.concatenate([s4, s4], axis=1)
    rel128 = jnp.zeros((D, D), jnp.float32).at[:relation_table.shape[0]].set(
        relation_table).astype(bf16)
    wc = (jnp.concatenate([W_ih_f[:, :D].T, W_ih_r[:, :D].T], axis=1)
          * s8).astype(bf16)
    wr = (jnp.concatenate([W_ih_f[:, D:].T, W_ih_r[:, D:].T], axis=1)
          * s8).astype(bf16)
    whf = (W_hh_f.T * s4).astype(bf16)
    whr = (W_hh_r.T * s4).astype(bf16)
    bf = (b_ih_f + b_hh_f)[None] * s4
    br = (b_ih_r + b_hh_r)[None] * s4

    return _encode(cpt_emb, meta, ht_vecs, rel128, wc, wr, whf, whr, bf, br,
                   W_hts.astype(bf16), b_hts[None], W_fc.astype(bf16),
                   b_fc[None])


# speedup vs baseline: 9.5560x; 1.0239x over previous
import functools

import jax
import jax.numpy as jnp
from jax import lax
from jax.experimental import pallas as pl
from jax.experimental.pallas import tpu as pltpu
from jax.experimental.pallas import tpu_sc as plsc

_NC = 2
_NS = 16
_NW = _NC * _NS
_CHUNK = 128


def _sc_gather_body(n_chunks, c1, idx2d_hbm, table_hbm, out1_hbm, out2_hbm,
                    idx_v, rows_v, sem):
    wid = lax.axis_index("s") * _NC + lax.axis_index("c")
    row0 = wid * n_chunks
    pltpu.sync_copy(idx2d_hbm.at[pl.ds(row0, n_chunks)], idx_v)

    def gather(j):
        b = lax.rem(j, 2)
        pltpu.async_copy(table_hbm.at[idx_v.at[j]], rows_v.at[b], sem)

    gather(0)

    def body(j, carry):
        @pl.when(j + 1 < n_chunks)
        def _():
            gather(j + 1)

        b = lax.rem(j, 2)
        pltpu.make_async_copy(table_hbm.at[idx_v.at[j]], rows_v.at[b],
                              sem).wait()
        c = row0 + j

        @pl.when(c < c1)
        def _():
            pltpu.sync_copy(rows_v.at[b],
                            out1_hbm.at[pl.ds(c * _CHUNK, _CHUNK)])

        @pl.when(c >= c1)
        def _():
            pltpu.sync_copy(rows_v.at[b],
                            out2_hbm.at[pl.ds((c - c1) * _CHUNK, _CHUNK)])

        return carry

    lax.fori_loop(0, n_chunks, body, 0)


def _sc_gather(idx_flat, table, n1):
    B = idx_flat.shape[0]
    D = table.shape[1]
    assert B % (_NW * _CHUNK) == 0 and n1 % _CHUNK == 0
    n_chunks = B // (_NW * _CHUNK)
    idx2d = idx_flat.reshape(-1, _CHUNK)
    mesh = plsc.VectorSubcoreMesh(core_axis_name="c", subcore_axis_name="s")
    k = functools.partial(
        pl.kernel,
        mesh=mesh,
        out_type=(
            jax.ShapeDtypeStruct((n1, D), table.dtype),
            jax.ShapeDtypeStruct((B - n1, D), table.dtype),
        ),
        scratch_types=[
            pltpu.VMEM((n_chunks, _CHUNK), jnp.int32),
            pltpu.VMEM((2, _CHUNK, D), table.dtype),
            pltpu.SemaphoreType.DMA,
        ],
    )(functools.partial(_sc_gather_body, n_chunks, n1 // _CHUNK))
    return k(idx2d, table)


def _enc_body(T, cpt_ref, meta_ref, ht_ref, rel128_ref, wc_ref, wr_ref,
              whf_ref, whr_ref, bias_ref, whts_ref, bhts_ref,
              wfc_ref, bfc_ref, out_ref):
    f32 = jnp.float32
    bf16 = jnp.bfloat16
    Bp = out_ref.shape[0]
    H = whf_ref.shape[0]

    rel_proj = (jnp.dot(rel128_ref[...], wr_ref[...],
                        preferred_element_type=f32)
                + bias_ref[...]).astype(bf16)
    wfull = jnp.concatenate([wc_ref[...], rel_proj], axis=0)

    iot = lax.broadcasted_iota(jnp.int32, (Bp, H), 1)
    ohs = []
    for t in range(T):
        r = meta_ref[t]
        r = jnp.where(r >= 17, r - 17, r)
        ohs.append((r[:, None] == iot) | (iot == H - 1))
    oh_all = jnp.concatenate(ohs, axis=0).astype(bf16)
    x_all = cpt_ref[...].reshape(T * Bp, H).astype(bf16)
    xcat = jnp.concatenate([x_all, oh_all], axis=1)
    gx_all = jnp.dot(xcat, wfull, preferred_element_type=f32)
    gx = [gx_all[t * Bp:(t + 1) * Bp] for t in range(T)]

    last = (jnp.clip(meta_ref[T], 1, T) - 1)[:, None]
    sel_mask = [(last == t).astype(f32) for t in range(T)]

    def sig2(x):
        return 0.5 * jnp.tanh(x) + 0.5

    def lstm(wh_ref, col0, order):
        h = jnp.zeros((Bp, H), f32)
        c = jnp.zeros((Bp, H), f32)
        sel = jnp.zeros((Bp, H), f32)
        for t in order:
            g = (gx[t][:, col0:col0 + 4 * H]
                 + jnp.dot(h.astype(bf16), wh_ref[...],
                           preferred_element_type=f32))
            i_g = sig2(g[:, 0:H])
            f_g = sig2(g[:, H:2 * H])
            g_g = jnp.tanh(g[:, 2 * H:3 * H])
            o_g = sig2(g[:, 3 * H:4 * H])
            c = f_g * c + i_g * g_g
            h = o_g * jnp.tanh(c)
            sel = sel + sel_mask[t] * h
        return sel

    sel_f = lstm(whf_ref, 0, range(T))
    sel_r = lstm(whr_ref, 4 * H, range(T - 1, -1, -1))

    def leaky(x):
        return jnp.where(x >= 0, x, 0.01 * x)

    hts = leaky(jnp.dot(ht_ref[...].astype(bf16), whts_ref[...],
                        preferred_element_type=f32) + bhts_ref[...])
    cat = jnp.concatenate([hts.astype(bf16), sel_f.astype(bf16),
                           sel_r.astype(bf16)], axis=1)
    fin = jnp.dot(cat, wfc_ref[...], preferred_element_type=f32) + bfc_ref[...]
    out_ref[...] = leaky(fin)


def _encode(cpt_emb, meta, ht_vecs, rel128, wc, wr, whf, whr, bias_mat,
            whts, bhts, wfc, bfc, block_p=512):
    T, P, D = cpt_emb.shape
    O = wfc.shape[1]
    grid = (P // block_p,)
    full = lambda shape: pl.BlockSpec(shape, lambda i: (0,) * len(shape))
    return pl.pallas_call(
        functools.partial(_enc_body, T),
        grid=grid,
        in_specs=[
            pl.BlockSpec((T, block_p, D), lambda i: (0, i, 0)),
            pl.BlockSpec((8, block_p), lambda i: (0, i)),
            pl.BlockSpec((block_p, 2 * D), lambda i: (i, 0)),
            full(rel128.shape), full(wc.shape), full(wr.shape),
            full(whf.shape), full(whr.shape), full(bias_mat.shape),
            full(whts.shape), full(bhts.shape),
            full(wfc.shape), full(bfc.shape),
        ],
        out_specs=pl.BlockSpec((block_p, O), lambda i: (i, 0)),
        out_shape=jax.ShapeDtypeStruct((P, O), jnp.float32),
        compiler_params=pltpu.CompilerParams(
            dimension_semantics=("arbitrary",)),
    )(cpt_emb, meta, ht_vecs, rel128, wc, wr, whf, whr, bias_mat,
      whts, bhts, wfc, bfc)


def kernel(concept_table, relation_table, W_ih_f, W_hh_f, b_ih_f, b_hh_f,
           W_ih_r, W_hh_r, b_ih_r, b_hh_r, W_hts, b_hts, W_fc, b_fc,
           cpt_paths, rel_paths, ht_ids, path_len):
    P, T = cpt_paths.shape
    V, D = concept_table.shape
    H = W_hh_f.shape[1]

    cpt_idx = cpt_paths.astype(jnp.int32).T.reshape(-1)
    ht_idx = ht_ids.astype(jnp.int32).reshape(-1)
    idx_all = jnp.concatenate([cpt_idx, ht_idx])
    cpt_rows, ht_rows = _sc_gather(idx_all, concept_table, T * P)
    cpt_emb = cpt_rows.reshape(T, P, D)
    ht_vecs = ht_rows.reshape(P, 2 * D)

    meta = jnp.concatenate([
        rel_paths.astype(jnp.int32).T,
        path_len.astype(jnp.int32)[None],
        jnp.zeros((8 - T - 1, P), jnp.int32),
    ], axis=0)
    bf16 = jnp.bfloat16
    s4 = jnp.concatenate([jnp.full((H,), 0.5), jnp.full((H,), 0.5),
                          jnp.ones((H,)), jnp.full((H,), 0.5)])[None]
    s8 = jnp.concatenate([s4, s4], axis=1)
    rel128 = jnp.zeros((D, D), jnp.float32).at[:relation_table.shape[0]].set(
        relation_table).astype(bf16)
    wc = (jnp.concatenate([W_ih_f[:, :D].T, W_ih_r[:, :D].T], axis=1)
          * s8).astype(bf16)
    wr = (jnp.concatenate([W_ih_f[:, D:].T, W_ih_r[:, D:].T], axis=1)
          * s8).astype(bf16)
    whf = (W_hh_f.T * s4).astype(bf16)
    whr = (W_hh_r.T * s4).astype(bf16)
    b8 = jnp.concatenate([(b_ih_f + b_hh_f)[None] * s4,
                          (b_ih_r + b_hh_r)[None] * s4], axis=1)
    bias_mat = jnp.zeros((D, 8 * H), jnp.float32).at[D - 1].set(b8[0])

    return _encode(cpt_emb, meta, ht_vecs, rel128, wc, wr, whf, whr, bias_mat,
                   W_hts.astype(bf16), b_hts[None], W_fc.astype(bf16),
                   b_fc[None])


# speedup vs baseline: 9.6680x; 1.0117x over previous
import functools

import jax
import jax.numpy as jnp
from jax import lax
from jax.experimental import pallas as pl
from jax.experimental.pallas import tpu as pltpu
from jax.experimental.pallas import tpu_sc as plsc

_NC = 2
_NS = 16
_NW = _NC * _NS
_CHUNK = 128


def _sc_gather_body(P, T, cpt_hbm, ht_hbm, table_hbm,
                    out_cpt_hbm, out_ht_hbm, idx_v, rows_v, sem, sem2):
    ppw = P // _NW
    npc = ppw // _CHUNK
    ncc = T * npc
    nhc = 2 * ppw // _CHUNK
    n_chunks = ncc + nhc
    wid = lax.axis_index("s") * _NC + lax.axis_index("c")
    p0 = wid * ppw

    def stage():
        for t in range(T):
            yield (cpt_hbm.at[pl.ds(t * P + p0, ppw)],
                   idx_v.at[pl.ds(t * ppw, ppw)])
        yield (ht_hbm.at[pl.ds(p0 * 2, ppw * 2)],
               idx_v.at[pl.ds(T * ppw, 2 * ppw)])

    for src, dst in stage():
        pltpu.async_copy(src, dst, sem2)
    for src, dst in stage():
        pltpu.make_async_copy(src, dst, sem2).wait()

    def gather(c):
        b = lax.rem(c, 2)
        pltpu.async_copy(table_hbm.at[idx_v.at[pl.ds(c * _CHUNK, _CHUNK)]],
                         rows_v.at[b], sem)

    gather(0)

    def body(c, carry):
        @pl.when(c + 1 < n_chunks)
        def _():
            gather(c + 1)

        b = lax.rem(c, 2)
        pltpu.make_async_copy(
            table_hbm.at[idx_v.at[pl.ds(c * _CHUNK, _CHUNK)]],
            rows_v.at[b], sem).wait()

        @pl.when(c < ncc)
        def _():
            t = c // npc
            pb = lax.rem(c, npc)
            pltpu.sync_copy(
                rows_v.at[b],
                out_cpt_hbm.at[pl.ds(t * P + p0 + pb * _CHUNK, _CHUNK)])

        @pl.when(c >= ncc)
        def _():
            pltpu.sync_copy(
                rows_v.at[b],
                out_ht_hbm.at[pl.ds(2 * p0 + (c - ncc) * _CHUNK, _CHUNK)])

        return carry

    lax.fori_loop(0, n_chunks, body, 0)


def _sc_gather(cpt_tm, ht_flat, table, P, T):
    D = table.shape[1]
    ppw = P // _NW
    assert ppw % _CHUNK == 0 and (T * ppw) % _CHUNK == 0
    mesh = plsc.VectorSubcoreMesh(core_axis_name="c", subcore_axis_name="s")
    k = functools.partial(
        pl.kernel,
        mesh=mesh,
        out_type=(
            jax.ShapeDtypeStruct((T * P, D), table.dtype),
            jax.ShapeDtypeStruct((2 * P, D), table.dtype),
        ),
        scratch_types=[
            pltpu.VMEM(((T + 2) * ppw,), jnp.int32),
            pltpu.VMEM((2, _CHUNK, D), table.dtype),
            pltpu.SemaphoreType.DMA,
            pltpu.SemaphoreType.DMA,
        ],
    )(functools.partial(_sc_gather_body, P, T))
    return k(cpt_tm, ht_flat, table)


def _enc_body(T, cpt_ref, meta_ref, ht_ref, rel128_ref, wc_ref, wr_ref,
              whf_ref, whr_ref, bias_ref, whts_ref, bhts_ref,
              wfc_ref, bfc_ref, out_ref):
    f32 = jnp.float32
    bf16 = jnp.bfloat16
    Bp = out_ref.shape[0]
    H = whf_ref.shape[0]

    rel_proj = (jnp.dot(rel128_ref[...], wr_ref[...],
                        preferred_element_type=f32)
                + bias_ref[...]).astype(bf16)
    wfull = jnp.concatenate([wc_ref[...], rel_proj], axis=0)

    iot = lax.broadcasted_iota(jnp.int32, (Bp, H), 1)
    ohs = []
    for t in range(T):
        r = meta_ref[t]
        r = jnp.where(r >= 17, r - 17, r)
        ohs.append((r[:, None] == iot) | (iot == H - 1))
    oh_all = jnp.concatenate(ohs, axis=0).astype(bf16)
    x_all = cpt_ref[...].reshape(T * Bp, H).astype(bf16)
    xcat = jnp.concatenate([x_all, oh_all], axis=1)
    gx_all = jnp.dot(xcat, wfull, preferred_element_type=f32)
    gx = [gx_all[t * Bp:(t + 1) * Bp] for t in range(T)]

    last = (jnp.clip(meta_ref[T], 1, T) - 1)[:, None]
    sel_mask = [(last == t).astype(f32) for t in range(T)]

    def sig2(x):
        return 0.5 * jnp.tanh(x) + 0.5

    def lstm(wh_ref, col0, order):
        h = jnp.zeros((Bp, H), f32)
        c = jnp.zeros((Bp, H), f32)
        sel = jnp.zeros((Bp, H), f32)
        for t in order:
            g = (gx[t][:, col0:col0 + 4 * H]
                 + jnp.dot(h.astype(bf16), wh_ref[...],
                           preferred_element_type=f32))
            i_g = sig2(g[:, 0:H])
            f_g = sig2(g[:, H:2 * H])
            g_g = jnp.tanh(g[:, 2 * H:3 * H])
            o_g = sig2(g[:, 3 * H:4 * H])
            c = f_g * c + i_g * g_g
            h = o_g * jnp.tanh(c)
            sel = sel + sel_mask[t] * h
        return sel

    sel_f = lstm(whf_ref, 0, range(T))
    sel_r = lstm(whr_ref, 4 * H, range(T - 1, -1, -1))

    def leaky(x):
        return jnp.where(x >= 0, x, 0.01 * x)

    hts = leaky(jnp.dot(ht_ref[...].astype(bf16), whts_ref[...],
                        preferred_element_type=f32) + bhts_ref[...])
    cat = jnp.concatenate([hts.astype(bf16), sel_f.astype(bf16),
                           sel_r.astype(bf16)], axis=1)
    fin = jnp.dot(cat, wfc_ref[...], preferred_element_type=f32) + bfc_ref[...]
    out_ref[...] = leaky(fin)


def _encode(cpt_emb, meta, ht_vecs, rel128, wc, wr, whf, whr, bias_mat,
            whts, bhts, wfc, bfc, block_p=512):
    T, P, D = cpt_emb.shape
    O = wfc.shape[1]
    grid = (P // block_p,)
    full = lambda shape: pl.BlockSpec(shape, lambda i: (0,) * len(shape))
    return pl.pallas_call(
        functools.partial(_enc_body, T),
        grid=grid,
        in_specs=[
            pl.BlockSpec((T, block_p, D), lambda i: (0, i, 0)),
            pl.BlockSpec((8, block_p), lambda i: (0, i)),
            pl.BlockSpec((block_p, 2 * D), lambda i: (i, 0)),
            full(rel128.shape), full(wc.shape), full(wr.shape),
            full(whf.shape), full(whr.shape), full(bias_mat.shape),
            full(whts.shape), full(bhts.shape),
            full(wfc.shape), full(bfc.shape),
        ],
        out_specs=pl.BlockSpec((block_p, O), lambda i: (i, 0)),
        out_shape=jax.ShapeDtypeStruct((P, O), jnp.float32),
        compiler_params=pltpu.CompilerParams(
            dimension_semantics=("arbitrary",)),
    )(cpt_emb, meta, ht_vecs, rel128, wc, wr, whf, whr, bias_mat,
      whts, bhts, wfc, bfc)


def kernel(concept_table, relation_table, W_ih_f, W_hh_f, b_ih_f, b_hh_f,
           W_ih_r, W_hh_r, b_ih_r, b_hh_r, W_hts, b_hts, W_fc, b_fc,
           cpt_paths, rel_paths, ht_ids, path_len):
    P, T = cpt_paths.shape
    V, D = concept_table.shape
    H = W_hh_f.shape[1]

    cpt_rows, ht_rows = _sc_gather(
        cpt_paths.astype(jnp.int32).T.reshape(-1),
        ht_ids.astype(jnp.int32).reshape(-1),
        concept_table, P, T)
    cpt_emb = cpt_rows.reshape(T, P, D)
    ht_vecs = ht_rows.reshape(P, 2 * D)

    meta = jnp.concatenate([
        rel_paths.astype(jnp.int32).T,
        path_len.astype(jnp.int32)[None],
        jnp.zeros((8 - T - 1, P), jnp.int32),
    ], axis=0)
    bf16 = jnp.bfloat16
    s4 = jnp.concatenate([jnp.full((H,), 0.5), jnp.full((H,), 0.5),
                          jnp.ones((H,)), jnp.full((H,), 0.5)])[None]
    s8 = jnp.concatenate([s4, s4], axis=1)
    nrel = relation_table.shape[0]
    rel128 = jnp.concatenate(
        [relation_table, jnp.zeros((D - nrel, D), jnp.float32)],
        axis=0).astype(bf16)
    wc = (jnp.concatenate([W_ih_f[:, :D].T, W_ih_r[:, :D].T], axis=1)
          * s8).astype(bf16)
    wr = (jnp.concatenate([W_ih_f[:, D:].T, W_ih_r[:, D:].T], axis=1)
          * s8).astype(bf16)
    whf = (W_hh_f.T * s4).astype(bf16)
    whr = (W_hh_r.T * s4).astype(bf16)
    b8 = jnp.concatenate([(b_ih_f + b_hh_f)[None] * s4,
                          (b_ih_r + b_hh_r)[None] * s4], axis=1)
    bias_mat = jnp.concatenate(
        [jnp.zeros((D - 1, 8 * H), jnp.float32), b8], axis=0)

    return _encode(cpt_emb, meta, ht_vecs, rel128, wc, wr, whf, whr, bias_mat,
                   W_hts.astype(bf16), b_hts[None], W_fc.astype(bf16),
                   b_fc[None])


# speedup vs baseline: 9.8344x; 1.0172x over previous
import functools

import jax
import jax.numpy as jnp
from jax import lax
from jax.experimental import pallas as pl
from jax.experimental.pallas import tpu as pltpu
from jax.experimental.pallas import tpu_sc as plsc

_NC = 2
_NS = 16
_NW = _NC * _NS
_CHUNK = 128


def _sc_gather_body(P, T, cpt_hbm, ht_hbm, table_hbm,
                    out_cpt_hbm, out_ht_hbm, idx_v, rows_v, sem, sem2):
    ppw = P // _NW
    npc = ppw // _CHUNK
    ncc = T * npc
    nhc = 2 * ppw // _CHUNK
    n_chunks = ncc + nhc
    wid = lax.axis_index("s") * _NC + lax.axis_index("c")
    p0 = wid * ppw

    def stage():
        for t in range(T):
            yield (cpt_hbm.at[pl.ds(t * P + p0, ppw)],
                   idx_v.at[pl.ds(t * ppw, ppw)])
        yield (ht_hbm.at[pl.ds(p0 * 2, ppw * 2)],
               idx_v.at[pl.ds(T * ppw, 2 * ppw)])

    for src, dst in stage():
        pltpu.async_copy(src, dst, sem2)
    for src, dst in stage():
        pltpu.make_async_copy(src, dst, sem2).wait()

    def gather(c):
        b = lax.rem(c, 2)
        pltpu.async_copy(table_hbm.at[idx_v.at[pl.ds(c * _CHUNK, _CHUNK)]],
                         rows_v.at[b], sem)

    gather(0)

    def body(c, carry):
        @pl.when(c + 1 < n_chunks)
        def _():
            gather(c + 1)

        b = lax.rem(c, 2)
        pltpu.make_async_copy(
            table_hbm.at[idx_v.at[pl.ds(c * _CHUNK, _CHUNK)]],
            rows_v.at[b], sem).wait()

        @pl.when(c < ncc)
        def _():
            t = c // npc
            pb = lax.rem(c, npc)
            pltpu.sync_copy(
                rows_v.at[b],
                out_cpt_hbm.at[pl.ds(t * P + p0 + pb * _CHUNK, _CHUNK)])

        @pl.when(c >= ncc)
        def _():
            pltpu.sync_copy(
                rows_v.at[b],
                out_ht_hbm.at[pl.ds(2 * p0 + (c - ncc) * _CHUNK, _CHUNK)])

        return carry

    lax.fori_loop(0, n_chunks, body, 0)


def _sc_gather(cpt_tm, ht_flat, table, P, T):
    D = table.shape[1]
    ppw = P // _NW
    assert ppw % _CHUNK == 0 and (T * ppw) % _CHUNK == 0
    mesh = plsc.VectorSubcoreMesh(core_axis_name="c", subcore_axis_name="s")
    k = functools.partial(
        pl.kernel,
        mesh=mesh,
        out_type=(
            jax.ShapeDtypeStruct((T * P, D), table.dtype),
            jax.ShapeDtypeStruct((2 * P, D), table.dtype),
        ),
        scratch_types=[
            pltpu.VMEM(((T + 2) * ppw,), jnp.int32),
            pltpu.VMEM((2, _CHUNK, D), table.dtype),
            pltpu.SemaphoreType.DMA,
            pltpu.SemaphoreType.DMA,
        ],
    )(functools.partial(_sc_gather_body, P, T))
    return k(cpt_tm, ht_flat, table)


def _enc_body(T, cpt_ref, meta_ref, ht_ref, rel128_ref, wc_ref, wr_ref,
              whf_ref, whr_ref, bias_ref, whts_ref, bhts_ref,
              wfc_ref, bfc_ref, out_ref):
    f32 = jnp.float32
    bf16 = jnp.bfloat16
    Bp = out_ref.shape[0]
    H = whf_ref.shape[0]

    rel_proj = (jnp.dot(rel128_ref[...], wr_ref[...],
                        preferred_element_type=f32)
                + bias_ref[...]).astype(bf16)
    wfull = jnp.concatenate([wc_ref[...], rel_proj], axis=0)

    iot = lax.broadcasted_iota(jnp.int32, (Bp, H), 1)
    ohs = []
    for t in range(T):
        r = meta_ref[:, t:t + 1]
        r = jnp.where(r >= 17, r - 17, r)
        ohs.append((r == iot) | (iot == H - 1))
    oh_all = jnp.concatenate(ohs, axis=0).astype(bf16)
    x_all = cpt_ref[...].reshape(T * Bp, H).astype(bf16)
    xcat = jnp.concatenate([x_all, oh_all], axis=1)
    gx_all = jnp.dot(xcat, wfull, preferred_element_type=f32)
    gx = [gx_all[t * Bp:(t + 1) * Bp] for t in range(T)]

    last = jnp.clip(meta_ref[:, T:T + 1], 1, T) - 1
    sel_mask = [(last == t).astype(f32) for t in range(T)]

    def sig2(x):
        return 0.5 * jnp.tanh(x) + 0.5

    def lstm(wh_ref, col0, order):
        h = jnp.zeros((Bp, H), f32)
        c = jnp.zeros((Bp, H), f32)
        sel = jnp.zeros((Bp, H), f32)
        for t in order:
            g = (gx[t][:, col0:col0 + 4 * H]
                 + jnp.dot(h.astype(bf16), wh_ref[...],
                           preferred_element_type=f32))
            i_g = sig2(g[:, 0:H])
            f_g = sig2(g[:, H:2 * H])
            g_g = jnp.tanh(g[:, 2 * H:3 * H])
            o_g = sig2(g[:, 3 * H:4 * H])
            c = f_g * c + i_g * g_g
            h = o_g * jnp.tanh(c)
            sel = sel + sel_mask[t] * h
        return sel

    sel_f = lstm(whf_ref, 0, range(T))
    sel_r = lstm(whr_ref, 4 * H, range(T - 1, -1, -1))

    def leaky(x):
        return jnp.where(x >= 0, x, 0.01 * x)

    hts = leaky(jnp.dot(ht_ref[...].astype(bf16), whts_ref[...],
                        preferred_element_type=f32) + bhts_ref[...])
    cat = jnp.concatenate([hts.astype(bf16), sel_f.astype(bf16),
                           sel_r.astype(bf16)], axis=1)
    fin = jnp.dot(cat, wfc_ref[...], preferred_element_type=f32) + bfc_ref[...]
    out_ref[...] = leaky(fin)


def _encode(cpt_emb, meta, ht_vecs, rel128, wc, wr, whf, whr, bias_mat,
            whts, bhts, wfc, bfc, block_p=1024):
    T, P, D = cpt_emb.shape
    O = wfc.shape[1]
    grid = (P // block_p,)
    full = lambda shape: pl.BlockSpec(shape, lambda i: (0,) * len(shape))
    return pl.pallas_call(
        functools.partial(_enc_body, T),
        grid=grid,
        in_specs=[
            pl.BlockSpec((T, block_p, D), lambda i: (0, i, 0)),
            pl.BlockSpec((block_p, 8), lambda i: (i, 0)),
            pl.BlockSpec((block_p, 2 * D), lambda i: (i, 0)),
            full(rel128.shape), full(wc.shape), full(wr.shape),
            full(whf.shape), full(whr.shape), full(bias_mat.shape),
            full(whts.shape), full(bhts.shape),
            full(wfc.shape), full(bfc.shape),
        ],
        out_specs=pl.BlockSpec((block_p, O), lambda i: (i, 0)),
        out_shape=jax.ShapeDtypeStruct((P, O), jnp.float32),
        compiler_params=pltpu.CompilerParams(
            dimension_semantics=("arbitrary",)),
    )(cpt_emb, meta, ht_vecs, rel128, wc, wr, whf, whr, bias_mat,
      whts, bhts, wfc, bfc)


def kernel(concept_table, relation_table, W_ih_f, W_hh_f, b_ih_f, b_hh_f,
           W_ih_r, W_hh_r, b_ih_r, b_hh_r, W_hts, b_hts, W_fc, b_fc,
           cpt_paths, rel_paths, ht_ids, path_len):
    P, T = cpt_paths.shape
    V, D = concept_table.shape
    H = W_hh_f.shape[1]

    cpt_rows, ht_rows = _sc_gather(
        cpt_paths.astype(jnp.int32).T.reshape(-1),
        ht_ids.astype(jnp.int32).reshape(-1),
        concept_table, P, T)
    cpt_emb = cpt_rows.reshape(T, P, D)
    ht_vecs = ht_rows.reshape(P, 2 * D)

    meta = jnp.concatenate([
        rel_paths.astype(jnp.int32),
        path_len.astype(jnp.int32)[:, None],
        jnp.zeros((P, 8 - T - 1), jnp.int32),
    ], axis=1)
    bf16 = jnp.bfloat16
    s4 = jnp.concatenate([jnp.full((H,), 0.5), jnp.full((H,), 0.5),
                          jnp.ones((H,)), jnp.full((H,), 0.5)])[None]
    s8 = jnp.concatenate([s4, s4], axis=1)
    nrel = relation_table.shape[0]
    rel128 = jnp.concatenate(
        [relation_table, jnp.zeros((D - nrel, D), jnp.float32)],
        axis=0).astype(bf16)
    wc = (jnp.concatenate([W_ih_f[:, :D].T, W_ih_r[:, :D].T], axis=1)
          * s8).astype(bf16)
    wr = (jnp.concatenate([W_ih_f[:, D:].T, W_ih_r[:, D:].T], axis=1)
          * s8).astype(bf16)
    whf = (W_hh_f.T * s4).astype(bf16)
    whr = (W_hh_r.T * s4).astype(bf16)
    b8 = jnp.concatenate([(b_ih_f + b_hh_f)[None] * s4,
                          (b_ih_r + b_hh_r)[None] * s4], axis=1)
    bias_mat = jnp.concatenate(
        [jnp.zeros((D - 1, 8 * H), jnp.float32), b8], axis=0)

    return _encode(cpt_emb, meta, ht_vecs, rel128, wc, wr, whf, whr, bias_mat,
                   W_hts.astype(bf16), b_hts[None], W_fc.astype(bf16),
                   b_fc[None])
